# baseline (speedup 1.0000x reference)
import numpy as np
import jax
import jax.numpy as jnp
from jax import lax
from jax.experimental import pallas as pl
from jax.experimental.pallas import tpu as pltpu

_N = 5000
_R = 8
_C = 640
_NP = _R * _C
_IOU_T = 0.1
_TWO_PI = 2.0 * np.pi
_PI = np.pi

_SB = 128


def _matcher_kernel(x_ref, y_ref, z_ref, dx_ref, dy_ref, dz_ref, r_ref, s_ref,
                    r_row_ref, s_row_ref, v8_ref, out_ref, segrow_ref,
                    geom_ref):
    idx = (lax.broadcasted_iota(jnp.int32, (_R, _C), 0) * _C
           + lax.broadcasted_iota(jnp.int32, (_R, _C), 1))
    valid = idx < _N

    x = x_ref[...]
    y = y_ref[...]
    z = z_ref[...]
    dx = dx_ref[...]
    dy = dy_ref[...]
    dz = dz_ref[...]
    r = r_ref[...]

    rr = r - jnp.floor(r / _TWO_PI + 0.5) * _TWO_PI
    cr = jnp.cos(rr)
    sr = jnp.sin(rr)
    ca = jnp.abs(cr)
    sa = jnp.abs(sr)
    hx = 0.5 * (dx * ca + dy * sa)
    hy = 0.5 * (dx * sa + dy * ca)
    x1 = x - hx
    x2 = x + hx
    y1 = y - hy
    y2 = y + hy
    z1 = z - 0.5 * dz
    z2 = z + 0.5 * dz
    vol = (x2 - x1) * (y2 - y1) * (z2 - z1)

    geoms = (x1, x2, y1, y2, z1, z2, vol)
    for k, g in enumerate(geoms):
        geom_ref[k * _R:(k + 1) * _R, :] = g

    lane = lax.broadcasted_iota(jnp.int32, (1, _C), 1)

    def _cond(st):
        _, _, _, nxt = st
        return nxt < _N

    def _body(st):
        idxm, seg, cnum, nxt = st
        rsub = nxt // _C
        csub = nxt - rsub * _C
        lm = lane == csub

        def ext(k):
            row = geom_ref[pl.ds(k * _R + rsub, 1), :]
            return jnp.sum(jnp.where(lm, row, 0.0))

        xx1 = ext(0)
        xx2 = ext(1)
        yy1 = ext(2)
        yy2 = ext(3)
        zz1 = ext(4)
        zz2 = ext(5)
        vv = ext(6)
        ix = jnp.maximum(jnp.minimum(x2, xx2) - jnp.maximum(x1, xx1), 0.0)
        iy = jnp.maximum(jnp.minimum(y2, yy2) - jnp.maximum(y1, yy1), 0.0)
        iz = jnp.maximum(jnp.minimum(z2, zz2) - jnp.maximum(z1, zz1), 0.0)
        inter = ix * iy * iz
        union = jnp.maximum(vol + vv - inter, 1e-6)
        mrow = jnp.logical_and(inter / union > _IOU_T, valid)
        idxm = jnp.where(mrow, _NP, idxm)
        seg = jnp.where(mrow, cnum, seg)
        return idxm, seg, cnum + 1, jnp.min(idxm)

    idxm0 = jnp.where(valid, idx, _NP)
    st0 = (idxm0, jnp.zeros((_R, _C), jnp.int32), jnp.int32(0), jnp.int32(0))
    _, seg, nseg, _ = lax.while_loop(_cond, _body, st0)

    for rrow in range(_R):
        segrow_ref[0:1, rrow * _C:(rrow + 1) * _C] = seg[rrow:rrow + 1, :]
    seg_row = segrow_ref[...]

    idx_row = lax.broadcasted_iota(jnp.int32, (1, _NP), 1)
    valid_row = idx_row < _N
    r_row = r_row_ref[...]
    s_row = s_row_ref[...]
    rr_row = r_row - jnp.floor(r_row / _TWO_PI + 0.5) * _TWO_PI
    sr_row = jnp.sin(rr_row)
    cr_row = jnp.cos(rr_row)
    v8 = v8_ref[...]
    lane16 = lax.broadcasted_iota(jnp.int32, (_SB, 16), 1)

    out_ref[...] = jnp.zeros((_NP, 16), jnp.float32)

    def _dot(a, b):
        return jax.lax.dot_general(a, b, (((1,), (0,)), ((), ())),
                                   preferred_element_type=jnp.float32)

    def _fcond(cb):
        return cb * _SB < nseg

    def _fbody(cb):
        base = cb * _SB
        cid = base + lax.broadcasted_iota(jnp.int32, (_SB, 1), 0)
        am = jnp.logical_and(seg_row == cid, valid_row)
        af = jnp.where(am, 1.0, 0.0)
        sm = jnp.where(am, s_row, 0.0)
        s1 = _dot(af, v8)
        cnt = s1[:, 0:1]
        sum_s = s1[:, 1:2]
        sd = _dot(sm, v8)
        max_s = jnp.max(sm, axis=1, keepdims=True)
        eqm = jnp.logical_and(am, s_row >= max_s)
        ridx = jnp.min(jnp.where(eqm, idx_row, _NP), axis=1, keepdims=True)
        ref_dir = jnp.sum(jnp.where(idx_row == ridx, rr_row, 0.0),
                          axis=1, keepdims=True)
        diff = jnp.abs(rr_row - ref_dir)
        diff = jnp.where(diff > _PI, _TWO_PI - diff, diff)
        m_a = diff > (_PI / 2.0)
        s_lt = jnp.sum(jnp.where(m_a, sm, 0.0), axis=1, keepdims=True)
        s_set = jnp.sum(jnp.where(m_a, 0.0, sm), axis=1, keepdims=True)
        flip_a = s_lt <= s_set
        q = jnp.where(jnp.logical_xor(m_a, flip_a), sm, -sm)
        sint = jnp.sum(q * sr_row, axis=1, keepdims=True)
        cost = jnp.sum(q * cr_row, axis=1, keepdims=True)
        theta = jnp.arctan2(sint, cost)
        inv_s = 1.0 / jnp.maximum(sum_s, 1e-12)
        vals = [sd[:, 2:3] * inv_s, sd[:, 3:4] * inv_s, sd[:, 4:5] * inv_s,
                sd[:, 5:6] * inv_s, sd[:, 6:7] * inv_s, sd[:, 7:8] * inv_s,
                theta, max_s, cnt]
        rows = jnp.zeros((_SB, 16), jnp.float32)
        for k, v in enumerate(vals):
            rows = jnp.where(lane16 == k, v, rows)
        rows = jnp.where(cnt > 0.0, rows, jnp.zeros((_SB, 16), jnp.float32))
        out_ref[pl.ds(base, _SB), :] = rows
        return cb + 1

    lax.while_loop(_fcond, _fbody, jnp.int32(0))


def _pad2d(v):
    return jnp.pad(v, (0, _NP - _N)).reshape(_R, _C)


@jax.jit
def kernel(boxes, scores):
    cols = [_pad2d(boxes[:, k]) for k in range(7)]
    sv = _pad2d(scores)
    r_row = jnp.pad(boxes[:, 6], (0, _NP - _N)).reshape(1, _NP)
    s_row = jnp.pad(scores, (0, _NP - _N)).reshape(1, _NP)
    ones = jnp.ones((_N,), jnp.float32)
    v8 = jnp.pad(
        jnp.stack([ones, scores, boxes[:, 0], boxes[:, 1], boxes[:, 2],
                   boxes[:, 3], boxes[:, 4], boxes[:, 5]], axis=1),
        ((0, _NP - _N), (0, 0)))
    out = pl.pallas_call(
        _matcher_kernel,
        out_shape=jax.ShapeDtypeStruct((_NP, 16), jnp.float32),
        scratch_shapes=[pltpu.VMEM((1, _NP), jnp.int32),
                        pltpu.VMEM((7 * _R, _C), jnp.float32)],
    )(*cols, sv, r_row, s_row, v8)
    boxes_fused = out[:_N, 0:7]
    scores_fused = out[:_N, 7]
    counts = out[:_N, 8]
    return boxes_fused, scores_fused, counts


# speedup vs baseline: 1.1469x; 1.1469x over previous
import numpy as np
import jax
import jax.numpy as jnp
from jax import lax
from jax.experimental import pallas as pl
from jax.experimental.pallas import tpu as pltpu

_N = 5000
_R = 8
_C = 640
_NP = _R * _C
_IOU_T = 0.1
_TWO_PI = 2.0 * np.pi
_PI = np.pi
_BIG = 3.0e38

_SB = 128


def _matcher_kernel(x_ref, y_ref, z_ref, dx_ref, dy_ref, dz_ref, r_ref, s_ref,
                    r_row_ref, s_row_ref, v8_ref, out_ref, segrow_ref):
    idx = (lax.broadcasted_iota(jnp.int32, (_R, _C), 0) * _C
           + lax.broadcasted_iota(jnp.int32, (_R, _C), 1))
    valid = idx < _N

    x = x_ref[...]
    y = y_ref[...]
    z = z_ref[...]
    dx = dx_ref[...]
    dy = dy_ref[...]
    dz = dz_ref[...]
    r = r_ref[...]

    rr = r - jnp.floor(r / _TWO_PI + 0.5) * _TWO_PI
    cr = jnp.cos(rr)
    sr = jnp.sin(rr)
    ca = jnp.abs(cr)
    sa = jnp.abs(sr)
    hx = 0.5 * (dx * ca + dy * sa)
    hy = 0.5 * (dx * sa + dy * ca)
    pad = jnp.where(valid, 0.0, _BIG)
    x1 = x - hx + pad
    x2 = x + hx + pad
    y1 = y - hy
    y2 = y + hy
    z1 = z - 0.5 * dz
    z2 = z + 0.5 * dz
    vol = (x2 - x1) * (y2 - y1) * (z2 - z1)

    def _bmin(a):
        return jnp.min(jnp.min(a, axis=1, keepdims=True), axis=0,
                       keepdims=True)

    def _bsum(m, a):
        t = jnp.where(m, a, 0.0)
        return jnp.sum(jnp.sum(t, axis=1, keepdims=True), axis=0,
                       keepdims=True)

    def _cond(st):
        _, _, nxt = st
        return nxt[0, 0] < _N

    def _body(st):
        a_arr, cnum, nxt = st
        mm = a_arr == nxt
        xx1 = _bsum(mm, x1)
        xx2 = _bsum(mm, x2)
        yy1 = _bsum(mm, y1)
        yy2 = _bsum(mm, y2)
        zz1 = _bsum(mm, z1)
        zz2 = _bsum(mm, z2)
        vv = _bsum(mm, vol)
        ix = jnp.maximum(jnp.minimum(x2, xx2) - jnp.maximum(x1, xx1), 0.0)
        iy = jnp.maximum(jnp.minimum(y2, yy2) - jnp.maximum(y1, yy1), 0.0)
        iz = jnp.maximum(jnp.minimum(z2, zz2) - jnp.maximum(z1, zz1), 0.0)
        inter = ix * iy * iz
        union = jnp.maximum(vol + vv - inter, 1e-6)
        mrow = inter / union > _IOU_T
        a_arr = jnp.where(mrow, _NP + cnum, a_arr)
        return a_arr, cnum + 1, _bmin(a_arr)

    a0 = jnp.where(valid, idx, _NP)
    a_fin, nseg, _ = lax.while_loop(_cond, _body,
                                    (a0, jnp.int32(0), _bmin(a0)))
    seg = a_fin - _NP

    for rrow in range(_R):
        segrow_ref[0:1, rrow * _C:(rrow + 1) * _C] = seg[rrow:rrow + 1, :]
    seg_row = segrow_ref[...]

    idx_row = lax.broadcasted_iota(jnp.int32, (1, _NP), 1)
    valid_row = idx_row < _N
    r_row = r_row_ref[...]
    s_row = s_row_ref[...]
    rr_row = r_row - jnp.floor(r_row / _TWO_PI + 0.5) * _TWO_PI
    sr_row = jnp.sin(rr_row)
    cr_row = jnp.cos(rr_row)
    v8 = v8_ref[...]
    lane16 = lax.broadcasted_iota(jnp.int32, (_SB, 16), 1)

    out_ref[...] = jnp.zeros((_NP, 16), jnp.float32)

    def _dot(a, b):
        return jax.lax.dot_general(a, b, (((1,), (0,)), ((), ())),
                                   preferred_element_type=jnp.float32)

    def _fcond(cb):
        return cb * _SB < nseg

    def _fbody(cb):
        base = cb * _SB
        cid = base + lax.broadcasted_iota(jnp.int32, (_SB, 1), 0)
        am = jnp.logical_and(seg_row == cid, valid_row)
        af = jnp.where(am, 1.0, 0.0)
        sm = jnp.where(am, s_row, 0.0)
        s1 = _dot(af, v8)
        cnt = s1[:, 0:1]
        sum_s = s1[:, 1:2]
        sd = _dot(sm, v8)
        max_s = jnp.max(sm, axis=1, keepdims=True)
        eqm = jnp.logical_and(am, s_row >= max_s)
        ridx = jnp.min(jnp.where(eqm, idx_row, _NP), axis=1, keepdims=True)
        ref_dir = jnp.sum(jnp.where(idx_row == ridx, rr_row, 0.0),
                          axis=1, keepdims=True)
        diff = jnp.abs(rr_row - ref_dir)
        diff = jnp.where(diff > _PI, _TWO_PI - diff, diff)
        m_a = diff > (_PI / 2.0)
        s_lt = jnp.sum(jnp.where(m_a, sm, 0.0), axis=1, keepdims=True)
        s_set = jnp.sum(jnp.where(m_a, 0.0, sm), axis=1, keepdims=True)
        flip_a = s_lt <= s_set
        q = jnp.where(jnp.logical_xor(m_a, flip_a), sm, -sm)
        sint = jnp.sum(q * sr_row, axis=1, keepdims=True)
        cost = jnp.sum(q * cr_row, axis=1, keepdims=True)
        theta = jnp.arctan2(sint, cost)
        inv_s = 1.0 / jnp.maximum(sum_s, 1e-12)
        vals = [sd[:, 2:3] * inv_s, sd[:, 3:4] * inv_s, sd[:, 4:5] * inv_s,
                sd[:, 5:6] * inv_s, sd[:, 6:7] * inv_s, sd[:, 7:8] * inv_s,
                theta, max_s, cnt]
        rows = jnp.zeros((_SB, 16), jnp.float32)
        for k, v in enumerate(vals):
            rows = jnp.where(lane16 == k, v, rows)
        rows = jnp.where(cnt > 0.0, rows, jnp.zeros((_SB, 16), jnp.float32))
        out_ref[pl.ds(base, _SB), :] = rows
        return cb + 1

    lax.while_loop(_fcond, _fbody, jnp.int32(0))


def _pad2d(v):
    return jnp.pad(v, (0, _NP - _N)).reshape(_R, _C)


@jax.jit
def kernel(boxes, scores):
    cols = [_pad2d(boxes[:, k]) for k in range(7)]
    sv = _pad2d(scores)
    r_row = jnp.pad(boxes[:, 6], (0, _NP - _N)).reshape(1, _NP)
    s_row = jnp.pad(scores, (0, _NP - _N)).reshape(1, _NP)
    ones = jnp.ones((_N,), jnp.float32)
    v8 = jnp.pad(
        jnp.stack([ones, scores, boxes[:, 0], boxes[:, 1], boxes[:, 2],
                   boxes[:, 3], boxes[:, 4], boxes[:, 5]], axis=1),
        ((0, _NP - _N), (0, 0)))
    out = pl.pallas_call(
        _matcher_kernel,
        out_shape=jax.ShapeDtypeStruct((_NP, 16), jnp.float32),
        scratch_shapes=[pltpu.VMEM((1, _NP), jnp.int32)],
    )(*cols, sv, r_row, s_row, v8)
    boxes_fused = out[:_N, 0:7]
    scores_fused = out[:_N, 7]
    counts = out[:_N, 8]
    return boxes_fused, scores_fused, counts


# speedup vs baseline: 1.1489x; 1.0018x over previous
import numpy as np
import jax
import jax.numpy as jnp
from jax import lax
from jax.experimental import pallas as pl
from jax.experimental.pallas import tpu as pltpu

_N = 5000
_R = 8
_C = 640
_NP = _R * _C
_IOU_T = 0.1
_TWO_PI = 2.0 * np.pi
_PI = np.pi
_BIG = 3.0e38

_SB = 128


def _matcher_kernel(x_ref, y_ref, z_ref, dx_ref, dy_ref, dz_ref, r_ref, s_ref,
                    r_row_ref, s_row_ref, v8_ref, out_ref, segrow_ref):
    idx = (lax.broadcasted_iota(jnp.int32, (_R, _C), 0) * _C
           + lax.broadcasted_iota(jnp.int32, (_R, _C), 1))
    valid = idx < _N

    x = x_ref[...]
    y = y_ref[...]
    z = z_ref[...]
    dx = dx_ref[...]
    dy = dy_ref[...]
    dz = dz_ref[...]
    r = r_ref[...]

    rr = r - jnp.floor(r / _TWO_PI + 0.5) * _TWO_PI
    cr = jnp.cos(rr)
    sr = jnp.sin(rr)
    ca = jnp.abs(cr)
    sa = jnp.abs(sr)
    hx = 0.5 * (dx * ca + dy * sa)
    hy = 0.5 * (dx * sa + dy * ca)
    pad = jnp.where(valid, 0.0, _BIG)
    x1 = x - hx + pad
    x2 = x + hx + pad
    y1 = y - hy
    y2 = y + hy
    z1 = z - 0.5 * dz
    z2 = z + 0.5 * dz
    vol = (x2 - x1) * (y2 - y1) * (z2 - z1)

    def _bmin(a):
        return jnp.min(jnp.min(a, axis=1, keepdims=True), axis=0,
                       keepdims=True)

    def _bsum(m, a):
        t = jnp.where(m, a, 0.0)
        return jnp.sum(jnp.sum(t, axis=1, keepdims=True), axis=0,
                       keepdims=True)

    def _cond(st):
        _, _, nxt = st
        return nxt[0, 0] < _N

    def _body(st):
        a_arr, cnum, nxt = st
        mm = a_arr == nxt
        xx1 = _bsum(mm, x1)
        xx2 = _bsum(mm, x2)
        yy1 = _bsum(mm, y1)
        yy2 = _bsum(mm, y2)
        zz1 = _bsum(mm, z1)
        zz2 = _bsum(mm, z2)
        vv = (xx2 - xx1) * (yy2 - yy1) * (zz2 - zz1)
        ix = jnp.maximum(jnp.minimum(x2, xx2) - jnp.maximum(x1, xx1), 0.0)
        iy = jnp.maximum(jnp.minimum(y2, yy2) - jnp.maximum(y1, yy1), 0.0)
        iz = jnp.maximum(jnp.minimum(z2, zz2) - jnp.maximum(z1, zz1), 0.0)
        inter = ix * iy * iz
        union = jnp.maximum(vol + vv - inter, 1e-6)
        mrow = inter / union > _IOU_T
        a_arr = jnp.where(mrow, _NP + cnum, a_arr)
        return a_arr, cnum + 1, _bmin(a_arr)

    a0 = jnp.where(valid, idx, _NP)
    a_fin, nseg, _ = lax.while_loop(_cond, _body,
                                    (a0, jnp.int32(0), _bmin(a0)))
    seg = a_fin - _NP

    for rrow in range(_R):
        segrow_ref[0:1, rrow * _C:(rrow + 1) * _C] = seg[rrow:rrow + 1, :]
    seg_row = segrow_ref[...]

    idx_row = lax.broadcasted_iota(jnp.int32, (1, _NP), 1)
    valid_row = idx_row < _N
    r_row = r_row_ref[...]
    s_row = s_row_ref[...]
    rr_row = r_row - jnp.floor(r_row / _TWO_PI + 0.5) * _TWO_PI
    sr_row = jnp.sin(rr_row)
    cr_row = jnp.cos(rr_row)
    v8 = v8_ref[...]
    lane16 = lax.broadcasted_iota(jnp.int32, (_SB, 16), 1)

    out_ref[...] = jnp.zeros((_NP, 16), jnp.float32)

    def _dot(a, b):
        return jax.lax.dot_general(a, b, (((1,), (0,)), ((), ())),
                                   preferred_element_type=jnp.float32)

    def _fcond(cb):
        return cb * _SB < nseg

    def _fbody(cb):
        base = cb * _SB
        cid = base + lax.broadcasted_iota(jnp.int32, (_SB, 1), 0)
        am = jnp.logical_and(seg_row == cid, valid_row)
        af = jnp.where(am, 1.0, 0.0)
        sm = jnp.where(am, s_row, 0.0)
        s1 = _dot(af, v8)
        cnt = s1[:, 0:1]
        sum_s = s1[:, 1:2]
        sd = _dot(sm, v8)
        max_s = jnp.max(sm, axis=1, keepdims=True)
        eqm = jnp.logical_and(am, s_row >= max_s)
        ridx = jnp.min(jnp.where(eqm, idx_row, _NP), axis=1, keepdims=True)
        ref_dir = jnp.sum(jnp.where(idx_row == ridx, rr_row, 0.0),
                          axis=1, keepdims=True)
        diff = jnp.abs(rr_row - ref_dir)
        diff = jnp.where(diff > _PI, _TWO_PI - diff, diff)
        m_a = diff > (_PI / 2.0)
        s_lt = jnp.sum(jnp.where(m_a, sm, 0.0), axis=1, keepdims=True)
        s_set = jnp.sum(jnp.where(m_a, 0.0, sm), axis=1, keepdims=True)
        flip_a = s_lt <= s_set
        q = jnp.where(jnp.logical_xor(m_a, flip_a), sm, -sm)
        sint = jnp.sum(q * sr_row, axis=1, keepdims=True)
        cost = jnp.sum(q * cr_row, axis=1, keepdims=True)
        theta = jnp.arctan2(sint, cost)
        inv_s = 1.0 / jnp.maximum(sum_s, 1e-12)
        vals = [sd[:, 2:3] * inv_s, sd[:, 3:4] * inv_s, sd[:, 4:5] * inv_s,
                sd[:, 5:6] * inv_s, sd[:, 6:7] * inv_s, sd[:, 7:8] * inv_s,
                theta, max_s, cnt]
        rows = jnp.zeros((_SB, 16), jnp.float32)
        for k, v in enumerate(vals):
            rows = jnp.where(lane16 == k, v, rows)
        rows = jnp.where(cnt > 0.0, rows, jnp.zeros((_SB, 16), jnp.float32))
        out_ref[pl.ds(base, _SB), :] = rows
        return cb + 1

    lax.while_loop(_fcond, _fbody, jnp.int32(0))


def _pad2d(v):
    return jnp.pad(v, (0, _NP - _N)).reshape(_R, _C)


@jax.jit
def kernel(boxes, scores):
    cols = [_pad2d(boxes[:, k]) for k in range(7)]
    sv = _pad2d(scores)
    r_row = jnp.pad(boxes[:, 6], (0, _NP - _N)).reshape(1, _NP)
    s_row = jnp.pad(scores, (0, _NP - _N)).reshape(1, _NP)
    ones = jnp.ones((_N,), jnp.float32)
    v8 = jnp.pad(
        jnp.stack([ones, scores, boxes[:, 0], boxes[:, 1], boxes[:, 2],
                   boxes[:, 3], boxes[:, 4], boxes[:, 5]], axis=1),
        ((0, _NP - _N), (0, 0)))
    out = pl.pallas_call(
        _matcher_kernel,
        out_shape=jax.ShapeDtypeStruct((_NP, 16), jnp.float32),
        scratch_shapes=[pltpu.VMEM((1, _NP), jnp.int32)],
    )(*cols, sv, r_row, s_row, v8)
    boxes_fused = out[:_N, 0:7]
    scores_fused = out[:_N, 7]
    counts = out[:_N, 8]
    return boxes_fused, scores_fused, counts
